# baseline (speedup 1.0000x reference)
import jax
import jax.numpy as jnp
from jax.experimental import pallas as pl

K = 20


def _conv2(x, w):
    return jnp.einsum('oc,bcnk->bonk', w, x)


def _conv1(x, w):
    return jnp.einsum('oc,bcn->bon', w, x)


def _bn_lrelu(x, g, b):
    sh = (1, x.shape[1]) + (1,) * (x.ndim - 2)
    scale = g / jnp.sqrt(1.0 + 1e-5)
    y = x * scale.reshape(sh) + b.reshape(sh)
    return jnp.where(y >= 0, y, 0.2 * y)


def _graph_feature(x, k):
    xt = jnp.transpose(x, (0, 2, 1))
    inner = jnp.einsum('bnc,bmc->bnm', xt, xt)
    sq = jnp.sum(xt * xt, axis=-1)
    pair = 2.0 * inner - sq[:, :, None] - sq[:, None, :]
    idx = jax.lax.top_k(jax.lax.stop_gradient(pair), k)[1]
    feat = jax.vmap(lambda pts, ii: pts[ii])(xt, idx)
    xc = xt[:, :, None, :]
    out = jnp.concatenate([feat - xc, jnp.broadcast_to(xc, feat.shape)], axis=-1)
    return jnp.transpose(out, (0, 3, 1, 2))


def kernel(x, l, p):
    b = x.shape[0]
    n = x.shape[2]
    f = _graph_feature(x, K)
    h = _bn_lrelu(_conv2(f, p['w1a']), p['g1a'], p['b1a'])
    h = _bn_lrelu(_conv2(h, p['w1b']), p['g1b'], p['b1b'])
    x1 = jnp.max(h, axis=-1)
    f = _graph_feature(x1, K)
    h = _bn_lrelu(_conv2(f, p['w2a']), p['g2a'], p['b2a'])
    h = _bn_lrelu(_conv2(h, p['w2b']), p['g2b'], p['b2b'])
    x2 = jnp.max(h, axis=-1)
    f = _graph_feature(x2, K)
    x3 = jnp.max(_bn_lrelu(_conv2(f, p['w3']), p['g3'], p['b3']), axis=-1)
    xc = jnp.concatenate([x1, x2, x3], axis=1)
    xg = _bn_lrelu(_conv1(xc, p['w4']), p['g4'], p['b4'])
    xmax = jnp.max(xg, axis=-1, keepdims=True)
    lv = l.reshape(b, -1, 1)
    lc = _bn_lrelu(_conv1(lv, p['wl']), p['gl'], p['bl'])
    comb = jnp.tile(jnp.concatenate([xmax, lc], axis=1), (1, 1, n))
    xf = jnp.concatenate([comb, x1, x2, x3], axis=1)
    xf = _bn_lrelu(_conv1(xf, p['w5']), p['g5'], p['b5'])
    xf = _bn_lrelu(_conv1(xf, p['w6']), p['g6'], p['b6'])
    xf = _bn_lrelu(_conv1(xf, p['w7']), p['g7'], p['b7'])
    return _conv1(xf, p['w8']) + p['b8'][None, :, None]


# speedup vs baseline: 9.5474x; 9.5474x over previous
import functools
import jax
import jax.numpy as jnp
from jax import lax
from jax.experimental import pallas as pl
from jax.experimental.pallas import tpu as pltpu
from jax.experimental.pallas import tpu_sc as plsc

KNN = 20
NEG = -3.0e38


def _lrelu(y):
    return jnp.where(y >= 0, y, 0.2 * y)


def _p_body(nglob, xb_ref, xf_ref, wat_ref, wct_ref,
            idx_ref, a_ref, c_ref):
    b = pl.program_id(0)
    xb = xb_ref[0]
    xf = xf_ref[0]
    rb = xb.shape[1]
    n = xf.shape[1]
    inner = lax.dot_general(xb, xf, (((0,), (0,)), ((), ())),
                            preferred_element_type=jnp.float32)
    sqf = jnp.sum(xf * xf, axis=0, keepdims=True)
    sqb = jnp.transpose(jnp.sum(xb * xb, axis=0, keepdims=True))
    a_ref[0] = lax.dot_general(xb, wat_ref[...], (((0,), (0,)), ((), ())),
                               preferred_element_type=jnp.float32)
    c_ref[0] = lax.dot_general(xb, wct_ref[...], (((0,), (0,)), ((), ())),
                               preferred_element_type=jnp.float32)

    pair = (2.0 * inner - sqb) - sqf
    iota_j = lax.broadcasted_iota(jnp.int32, (rb, n), 1)
    lane_t = lax.broadcasted_iota(jnp.int32, (rb, 128), 1)
    acc = jnp.zeros((rb, 128), jnp.int32)
    base = b * nglob
    for t in range(KNN):
        m = jnp.max(pair, axis=1, keepdims=True)
        cand = jnp.where(pair == m, iota_j, n)
        j = jnp.min(cand, axis=1, keepdims=True)
        acc = jnp.where(lane_t == t,
                        jnp.broadcast_to(j + base, (rb, 128)), acc)
        pair = jnp.where(iota_j == j, NEG, pair)
    idx_ref[0] = acc


def _run_p(x, wat, wct, rb=128):
    bsz, ch, n = x.shape
    xb_spec = pl.BlockSpec((1, ch, rb), lambda b, r: (b, 0, r))
    xf_spec = pl.BlockSpec((1, ch, n), lambda b, r: (b, 0, 0))
    grid = (bsz, n // rb)
    out_shapes = (
        jax.ShapeDtypeStruct((bsz, n, 128), jnp.int32),
        jax.ShapeDtypeStruct((bsz, n, 128), jnp.float32),
        jax.ShapeDtypeStruct((bsz, n, 64), jnp.float32),
    )
    out_specs = (
        pl.BlockSpec((1, rb, 128), lambda b, r: (b, r, 0)),
        pl.BlockSpec((1, rb, 128), lambda b, r: (b, r, 0)),
        pl.BlockSpec((1, rb, 64), lambda b, r: (b, r, 0)),
    )
    wa_spec = pl.BlockSpec(wat.shape, lambda b, r: (0, 0))
    wc_spec = pl.BlockSpec(wct.shape, lambda b, r: (0, 0))
    return pl.pallas_call(
        functools.partial(_p_body, n),
        grid=grid,
        in_specs=[xb_spec, xf_spec, wa_spec, wc_spec],
        out_specs=out_specs,
        out_shape=out_shapes,
    )(x, x, wat, wct)


def _sc_gather(table, idx_flat):
    e_total = idx_flat.shape[0]
    info = plsc.get_sparse_core_info()
    nw = info.num_cores * info.num_subcores
    per_w = e_total // nw
    chunk = 128
    nchunk = per_w // chunk
    mesh = plsc.VectorSubcoreMesh(core_axis_name="c", subcore_axis_name="s")

    @functools.partial(
        pl.kernel, mesh=mesh,
        out_type=jax.ShapeDtypeStruct((e_total, 128), jnp.float32),
        scratch_types=[
            pltpu.VMEM((chunk,), jnp.int32),
            pltpu.VMEM((chunk, 128), jnp.float32),
            pltpu.SemaphoreType.DMA,
        ],
    )
    def gk(table_hbm, idx_hbm, out_hbm, idx_v, rows_v, sem):
        wid = lax.axis_index("s") * info.num_cores + lax.axis_index("c")
        wbase = wid * per_w

        def body(ci, carry):
            base = wbase + ci * chunk
            pltpu.sync_copy(idx_hbm.at[pl.ds(base, chunk)], idx_v)
            pltpu.async_copy(table_hbm.at[idx_v], rows_v, sem).wait()
            pltpu.sync_copy(rows_v, out_hbm.at[pl.ds(base, chunk)])
            return carry

        lax.fori_loop(0, nchunk, body, 0)

    return gk(table, idx_flat)


def _q_body(g_ref, c_ref, sa_ref, ba_ref, wbt_ref, sb_ref, bb_ref,
            out_ref, xt_ref):
    c = c_ref[0]
    sa = sa_ref[...]
    ba = ba_ref[...]
    hs = [
        _lrelu((g_ref[0, k][:, :64] + c) * sa + ba)
        for k in range(KNN)
    ]
    hcat = jnp.concatenate(hs, axis=0)
    y = jnp.dot(hcat, wbt_ref[...], preferred_element_type=jnp.float32)
    z = _lrelu(y * sb_ref[...] + bb_ref[...])
    rb = c.shape[0]
    acc = z[0:rb]
    for k in range(1, KNN):
        acc = jnp.maximum(acc, z[k * rb:(k + 1) * rb])
    out_ref[0] = acc
    xt_ref[0] = jnp.transpose(acc)


def _run_q(gath, cmat, sa, ba, wbt, sb, bb, rb=256):
    bsz, _, n, _ = gath.shape
    grid = (bsz, n // rb)
    small = lambda arr: pl.BlockSpec(arr.shape, lambda b, r: (0, 0))
    return pl.pallas_call(
        _q_body,
        grid=grid,
        in_specs=[
            pl.BlockSpec((1, KNN, rb, 128), lambda b, r: (b, 0, r, 0)),
            pl.BlockSpec((1, rb, 64), lambda b, r: (b, r, 0)),
            small(sa), small(ba), small(wbt), small(sb), small(bb),
        ],
        out_specs=(
            pl.BlockSpec((1, rb, 64), lambda b, r: (b, r, 0)),
            pl.BlockSpec((1, 64, rb), lambda b, r: (b, 0, r)),
        ),
        out_shape=(
            jax.ShapeDtypeStruct((bsz, n, 64), jnp.float32),
            jax.ShapeDtypeStruct((bsz, 64, n), jnp.float32),
        ),
    )(gath, cmat, sa, ba, wbt, sb, bb)


def _qr1_body(g_ref, c_ref, s3_ref, b3_ref, x1_ref, x2_ref, w4t_ref,
              s4_ref, b4_ref, x3_ref, xmax_ref):
    r = pl.program_id(1)
    mx = g_ref[0, 0][:, :64]
    for k in range(1, KNN):
        mx = jnp.maximum(mx, g_ref[0, k][:, :64])
    x3 = _lrelu((mx + c_ref[0]) * s3_ref[...] + b3_ref[...])
    x3_ref[0] = x3
    xc = jnp.concatenate([x1_ref[0], x2_ref[0], x3], axis=1)
    xg = _lrelu(jnp.dot(xc, w4t_ref[...], preferred_element_type=jnp.float32) * s4_ref[...] + b4_ref[...])
    part = jnp.max(xg, axis=0, keepdims=True)

    @pl.when(r == 0)
    def _():
        xmax_ref[0] = jnp.full((1, 1024), NEG, jnp.float32)

    xmax_ref[0] = jnp.maximum(xmax_ref[0], part)


def _run_qr1(gath3, c3, s3, b3, x1, x2, w4t, s4, b4, rb=256):
    bsz, _, n, _ = gath3.shape
    grid = (bsz, n // rb)
    small = lambda arr: pl.BlockSpec(arr.shape, lambda b, r: (0, 0))
    return pl.pallas_call(
        _qr1_body,
        grid=grid,
        in_specs=[
            pl.BlockSpec((1, KNN, rb, 128), lambda b, r: (b, 0, r, 0)),
            pl.BlockSpec((1, rb, 64), lambda b, r: (b, r, 0)),
            small(s3), small(b3),
            pl.BlockSpec((1, rb, 64), lambda b, r: (b, r, 0)),
            pl.BlockSpec((1, rb, 64), lambda b, r: (b, r, 0)),
            small(w4t), small(s4), small(b4),
        ],
        out_specs=(
            pl.BlockSpec((1, rb, 64), lambda b, r: (b, r, 0)),
            pl.BlockSpec((1, 1, 1024), lambda b, r: (b, 0, 0)),
        ),
        out_shape=(
            jax.ShapeDtypeStruct((bsz, n, 64), jnp.float32),
            jax.ShapeDtypeStruct((bsz, 1, 1024), jnp.float32),
        ),
    )(gath3, c3, s3, b3, x1, x2, w4t, s4, b4)


def _r2_body(xmax_ref, l_ref, wlt_ref, sl_ref, bl_ref, w5at_ref,
             x1_ref, x2_ref, x3_ref, w5bt_ref, s5_ref, b5_ref,
             w6t_ref, s6_ref, b6_ref, w7t_ref, s7_ref, b7_ref,
             w8_ref, b8_ref, out_ref):
    lc = _lrelu(jnp.dot(l_ref[0], wlt_ref[...],
                        preferred_element_type=jnp.float32) * sl_ref[...] + bl_ref[...])
    cin = jnp.concatenate([xmax_ref[0], lc], axis=1)
    const5 = jnp.dot(cin, w5at_ref[...], preferred_element_type=jnp.float32)
    xp = jnp.concatenate([x1_ref[0], x2_ref[0], x3_ref[0]], axis=1)
    y5 = _lrelu((jnp.dot(xp, w5bt_ref[...], preferred_element_type=jnp.float32) + const5) * s5_ref[...] + b5_ref[...])
    y6 = _lrelu(jnp.dot(y5, w6t_ref[...], preferred_element_type=jnp.float32) * s6_ref[...] + b6_ref[...])
    y7 = _lrelu(jnp.dot(y6, w7t_ref[...], preferred_element_type=jnp.float32) * s7_ref[...] + b7_ref[...])
    outt = lax.dot_general(w8_ref[...], y7, (((1,), (1,)), ((), ())),
                           preferred_element_type=jnp.float32)
    out_ref[0] = outt + b8_ref[...]


def _run_r2(xmax, l2, wlt, sl, bl, w5at, x1, x2, x3, w5bt, s5, b5,
            w6t, s6, b6, w7t, s7, b7, w8, b8c, rb=256):
    bsz, n, _ = x1.shape
    grid = (bsz, n // rb)
    small = lambda arr: pl.BlockSpec(arr.shape, lambda b, r: tuple(0 for _ in arr.shape))
    xspec = pl.BlockSpec((1, rb, 64), lambda b, r: (b, r, 0))
    return pl.pallas_call(
        _r2_body,
        grid=grid,
        in_specs=[
            pl.BlockSpec((1, 1, 1024), lambda b, r: (b, 0, 0)),
            pl.BlockSpec((1, 1, 16), lambda b, r: (b, 0, 0)),
            small(wlt), small(sl), small(bl), small(w5at),
            xspec, xspec, xspec,
            small(w5bt), small(s5), small(b5),
            small(w6t), small(s6), small(b6),
            small(w7t), small(s7), small(b7),
            small(w8), small(b8c),
        ],
        out_specs=pl.BlockSpec((1, 50, rb), lambda b, r: (b, 0, r)),
        out_shape=jax.ShapeDtypeStruct((bsz, 50, n), jnp.float32),
    )(xmax, l2, wlt, sl, bl, w5at, x1, x2, x3, w5bt, s5, b5,
      w6t, s6, b6, w7t, s7, b7, w8, b8c)


def _scale(g):
    return (g / jnp.sqrt(1.0 + 1e-5))[None, :]


def kernel(x, l, p):
    bsz, _, n = x.shape
    edges = bsz * KNN * n

    def stage_gather(xin, wa, wb):
        wat = jnp.pad(jnp.transpose(wa), ((0, 0), (0, 64)))
        wct = jnp.transpose(wb - wa)
        idx, a, c = _run_p(xin, wat, wct)
        idx_knm = jnp.transpose(idx[:, :, :KNN], (0, 2, 1)).reshape(edges)
        gath = _sc_gather(a.reshape(bsz * n, 128), idx_knm)
        return gath.reshape(bsz, KNN, n, 128), c

    g1, c1 = stage_gather(x, p['w1a'][:, :3], p['w1a'][:, 3:])
    x1, x1t = _run_q(g1, c1, _scale(p['g1a']), p['b1a'][None, :],
                     jnp.transpose(p['w1b']), _scale(p['g1b']),
                     p['b1b'][None, :])
    g2, c2 = stage_gather(x1t, p['w2a'][:, :64], p['w2a'][:, 64:])
    x2, x2t = _run_q(g2, c2, _scale(p['g2a']), p['b2a'][None, :],
                     jnp.transpose(p['w2b']), _scale(p['g2b']),
                     p['b2b'][None, :])
    g3, c3 = stage_gather(x2t, p['w3'][:, :64], p['w3'][:, 64:])
    x3, xmax = _run_qr1(g3, c3, _scale(p['g3']), p['b3'][None, :], x1, x2,
                        jnp.transpose(p['w4']), _scale(p['g4']),
                        p['b4'][None, :])
    out = _run_r2(xmax, l.reshape(bsz, 1, 16), jnp.transpose(p['wl']),
                  _scale(p['gl']), p['bl'][None, :],
                  jnp.transpose(p['w5'][:, :1088]),
                  x1, x2, x3, jnp.transpose(p['w5'][:, 1088:]),
                  _scale(p['g5']), p['b5'][None, :],
                  jnp.transpose(p['w6']), _scale(p['g6']), p['b6'][None, :],
                  jnp.transpose(p['w7']), _scale(p['g7']), p['b7'][None, :],
                  p['w8'], p['b8'][:, None])
    return out


# speedup vs baseline: 14.1735x; 1.4845x over previous
import functools
import jax
import jax.numpy as jnp
from jax import lax
from jax.experimental import pallas as pl
from jax.experimental.pallas import tpu as pltpu
from jax.experimental.pallas import tpu_sc as plsc

KNN = 20
NEG = -3.0e38


def _lrelu(y):
    return jnp.where(y >= 0, y, 0.2 * y)


def _p_body(nglob, xb_ref, xf_ref, wat_ref, wct_ref,
            idx_ref, a_ref, c_ref):
    b = pl.program_id(0)
    xb = xb_ref[0]
    xf = xf_ref[0]
    rb = xb.shape[1]
    n = xf.shape[1]
    inner = lax.dot_general(xb, xf, (((0,), (0,)), ((), ())),
                            preferred_element_type=jnp.float32)
    sqf = jnp.sum(xf * xf, axis=0, keepdims=True)
    sqb = jnp.transpose(jnp.sum(xb * xb, axis=0, keepdims=True))
    a_ref[0] = lax.dot_general(xb, wat_ref[...], (((0,), (0,)), ((), ())),
                               preferred_element_type=jnp.float32)
    c_ref[0] = lax.dot_general(xb, wct_ref[...], (((0,), (0,)), ((), ())),
                               preferred_element_type=jnp.float32)

    pair = (2.0 * inner - sqb) - sqf
    iota_j = lax.broadcasted_iota(jnp.int32, (rb, n), 1)
    lane_t = lax.broadcasted_iota(jnp.int32, (rb, 128), 1)
    acc = jnp.zeros((rb, 128), jnp.int32)
    base = b * nglob
    for t in range(KNN):
        m = jnp.max(pair, axis=1, keepdims=True)
        eqm = pair == m
        j = jnp.min(jnp.where(eqm, iota_j, n), axis=1, keepdims=True)
        acc = jnp.where(lane_t == t,
                        jnp.broadcast_to(j + base, (rb, 128)), acc)
        pair = jnp.where(eqm, NEG, pair)
    idx_ref[0] = acc


def _run_p(x, wat, wct, rb=128):
    bsz, ch, n = x.shape
    xb_spec = pl.BlockSpec((1, ch, rb), lambda b, r: (b, 0, r))
    xf_spec = pl.BlockSpec((1, ch, n), lambda b, r: (b, 0, 0))
    grid = (bsz, n // rb)
    out_shapes = (
        jax.ShapeDtypeStruct((bsz, n, 128), jnp.int32),
        jax.ShapeDtypeStruct((bsz, n, 128), jnp.float32),
        jax.ShapeDtypeStruct((bsz, n, 64), jnp.float32),
    )
    out_specs = (
        pl.BlockSpec((1, rb, 128), lambda b, r: (b, r, 0)),
        pl.BlockSpec((1, rb, 128), lambda b, r: (b, r, 0)),
        pl.BlockSpec((1, rb, 64), lambda b, r: (b, r, 0)),
    )
    wa_spec = pl.BlockSpec(wat.shape, lambda b, r: (0, 0))
    wc_spec = pl.BlockSpec(wct.shape, lambda b, r: (0, 0))
    return pl.pallas_call(
        functools.partial(_p_body, n),
        grid=grid,
        in_specs=[xb_spec, xf_spec, wa_spec, wc_spec],
        out_specs=out_specs,
        out_shape=out_shapes,
    )(x, x, wat, wct)


def _sc_gather(table, idx_flat):
    e_total = idx_flat.shape[0]
    info = plsc.get_sparse_core_info()
    nw = info.num_cores * info.num_subcores
    per_w = e_total // nw
    chunk = 128
    nchunk = per_w // chunk
    mesh = plsc.VectorSubcoreMesh(core_axis_name="c", subcore_axis_name="s")

    @functools.partial(
        pl.kernel, mesh=mesh,
        out_type=jax.ShapeDtypeStruct((e_total, 128), jnp.float32),
        scratch_types=[
            pltpu.VMEM((chunk,), jnp.int32),
            pltpu.VMEM((chunk, 128), jnp.float32),
            pltpu.SemaphoreType.DMA,
        ],
    )
    def gk(table_hbm, idx_hbm, out_hbm, idx_v, rows_v, sem):
        wid = lax.axis_index("s") * info.num_cores + lax.axis_index("c")
        wbase = wid * per_w

        def body(ci, carry):
            base = wbase + ci * chunk
            pltpu.sync_copy(idx_hbm.at[pl.ds(base, chunk)], idx_v)
            pltpu.async_copy(table_hbm.at[idx_v], rows_v, sem).wait()
            pltpu.sync_copy(rows_v, out_hbm.at[pl.ds(base, chunk)])
            return carry

        lax.fori_loop(0, nchunk, body, 0)

    return gk(table, idx_flat)


def _q_body(g_ref, c_ref, sa_ref, ba_ref, wbt_ref, sb_ref, bb_ref,
            out_ref, xt_ref):
    c = c_ref[0]
    sa = sa_ref[...]
    ba = ba_ref[...]
    hs = [
        _lrelu((g_ref[0, k][:, :64] + c) * sa + ba)
        for k in range(KNN)
    ]
    hcat = jnp.concatenate(hs, axis=0)
    y = jnp.dot(hcat, wbt_ref[...], preferred_element_type=jnp.float32)
    z = _lrelu(y * sb_ref[...] + bb_ref[...])
    rb = c.shape[0]
    acc = z[0:rb]
    for k in range(1, KNN):
        acc = jnp.maximum(acc, z[k * rb:(k + 1) * rb])
    out_ref[0] = acc
    xt_ref[0] = jnp.transpose(acc)


def _run_q(gath, cmat, sa, ba, wbt, sb, bb, rb=256):
    bsz, _, n, _ = gath.shape
    grid = (bsz, n // rb)
    small = lambda arr: pl.BlockSpec(arr.shape, lambda b, r: (0, 0))
    return pl.pallas_call(
        _q_body,
        grid=grid,
        in_specs=[
            pl.BlockSpec((1, KNN, rb, 128), lambda b, r: (b, 0, r, 0)),
            pl.BlockSpec((1, rb, 64), lambda b, r: (b, r, 0)),
            small(sa), small(ba), small(wbt), small(sb), small(bb),
        ],
        out_specs=(
            pl.BlockSpec((1, rb, 64), lambda b, r: (b, r, 0)),
            pl.BlockSpec((1, 64, rb), lambda b, r: (b, 0, r)),
        ),
        out_shape=(
            jax.ShapeDtypeStruct((bsz, n, 64), jnp.float32),
            jax.ShapeDtypeStruct((bsz, 64, n), jnp.float32),
        ),
    )(gath, cmat, sa, ba, wbt, sb, bb)


def _qr1_body(g_ref, c_ref, s3_ref, b3_ref, x1_ref, x2_ref, w4t_ref,
              s4_ref, b4_ref, x3_ref, xmax_ref):
    r = pl.program_id(1)
    mx = g_ref[0, 0][:, :64]
    for k in range(1, KNN):
        mx = jnp.maximum(mx, g_ref[0, k][:, :64])
    x3 = _lrelu((mx + c_ref[0]) * s3_ref[...] + b3_ref[...])
    x3_ref[0] = x3
    xc = jnp.concatenate([x1_ref[0], x2_ref[0], x3], axis=1)
    xg = _lrelu(jnp.dot(xc, w4t_ref[...], preferred_element_type=jnp.float32) * s4_ref[...] + b4_ref[...])
    part = jnp.max(xg, axis=0, keepdims=True)

    @pl.when(r == 0)
    def _():
        xmax_ref[0] = jnp.full((1, 1024), NEG, jnp.float32)

    xmax_ref[0] = jnp.maximum(xmax_ref[0], part)


def _run_qr1(gath3, c3, s3, b3, x1, x2, w4t, s4, b4, rb=256):
    bsz, _, n, _ = gath3.shape
    grid = (bsz, n // rb)
    small = lambda arr: pl.BlockSpec(arr.shape, lambda b, r: (0, 0))
    return pl.pallas_call(
        _qr1_body,
        grid=grid,
        in_specs=[
            pl.BlockSpec((1, KNN, rb, 128), lambda b, r: (b, 0, r, 0)),
            pl.BlockSpec((1, rb, 64), lambda b, r: (b, r, 0)),
            small(s3), small(b3),
            pl.BlockSpec((1, rb, 64), lambda b, r: (b, r, 0)),
            pl.BlockSpec((1, rb, 64), lambda b, r: (b, r, 0)),
            small(w4t), small(s4), small(b4),
        ],
        out_specs=(
            pl.BlockSpec((1, rb, 64), lambda b, r: (b, r, 0)),
            pl.BlockSpec((1, 1, 1024), lambda b, r: (b, 0, 0)),
        ),
        out_shape=(
            jax.ShapeDtypeStruct((bsz, n, 64), jnp.float32),
            jax.ShapeDtypeStruct((bsz, 1, 1024), jnp.float32),
        ),
    )(gath3, c3, s3, b3, x1, x2, w4t, s4, b4)


def _r2_body(xmax_ref, l_ref, wlt_ref, sl_ref, bl_ref, w5at_ref,
             x1_ref, x2_ref, x3_ref, w5bt_ref, s5_ref, b5_ref,
             w6t_ref, s6_ref, b6_ref, w7t_ref, s7_ref, b7_ref,
             w8_ref, b8_ref, out_ref):
    lc = _lrelu(jnp.dot(l_ref[0], wlt_ref[...],
                        preferred_element_type=jnp.float32) * sl_ref[...] + bl_ref[...])
    cin = jnp.concatenate([xmax_ref[0], lc], axis=1)
    const5 = jnp.dot(cin, w5at_ref[...], preferred_element_type=jnp.float32)
    xp = jnp.concatenate([x1_ref[0], x2_ref[0], x3_ref[0]], axis=1)
    y5 = _lrelu((jnp.dot(xp, w5bt_ref[...], preferred_element_type=jnp.float32) + const5) * s5_ref[...] + b5_ref[...])
    y6 = _lrelu(jnp.dot(y5, w6t_ref[...], preferred_element_type=jnp.float32) * s6_ref[...] + b6_ref[...])
    y7 = _lrelu(jnp.dot(y6, w7t_ref[...], preferred_element_type=jnp.float32) * s7_ref[...] + b7_ref[...])
    outt = lax.dot_general(w8_ref[...], y7, (((1,), (1,)), ((), ())),
                           preferred_element_type=jnp.float32)
    out_ref[0] = outt + b8_ref[...]


def _run_r2(xmax, l2, wlt, sl, bl, w5at, x1, x2, x3, w5bt, s5, b5,
            w6t, s6, b6, w7t, s7, b7, w8, b8c, rb=256):
    bsz, n, _ = x1.shape
    grid = (bsz, n // rb)
    small = lambda arr: pl.BlockSpec(arr.shape, lambda b, r: tuple(0 for _ in arr.shape))
    xspec = pl.BlockSpec((1, rb, 64), lambda b, r: (b, r, 0))
    return pl.pallas_call(
        _r2_body,
        grid=grid,
        in_specs=[
            pl.BlockSpec((1, 1, 1024), lambda b, r: (b, 0, 0)),
            pl.BlockSpec((1, 1, 16), lambda b, r: (b, 0, 0)),
            small(wlt), small(sl), small(bl), small(w5at),
            xspec, xspec, xspec,
            small(w5bt), small(s5), small(b5),
            small(w6t), small(s6), small(b6),
            small(w7t), small(s7), small(b7),
            small(w8), small(b8c),
        ],
        out_specs=pl.BlockSpec((1, 50, rb), lambda b, r: (b, 0, r)),
        out_shape=jax.ShapeDtypeStruct((bsz, 50, n), jnp.float32),
    )(xmax, l2, wlt, sl, bl, w5at, x1, x2, x3, w5bt, s5, b5,
      w6t, s6, b6, w7t, s7, b7, w8, b8c)


def _scale(g):
    return (g / jnp.sqrt(1.0 + 1e-5))[None, :]


def kernel(x, l, p):
    half = x.shape[0] // 2
    return jnp.concatenate([
        _forward(x[:half], l[:half], p),
        _forward(x[half:], l[half:], p),
    ], axis=0)


def _forward(x, l, p):
    bsz, _, n = x.shape
    edges = bsz * KNN * n

    def stage_gather(xin, wa, wb):
        wat = jnp.pad(jnp.transpose(wa), ((0, 0), (0, 64)))
        wct = jnp.transpose(wb - wa)
        idx, a, c = _run_p(xin, wat, wct)
        idx_knm = jnp.transpose(idx[:, :, :KNN], (0, 2, 1)).reshape(edges)
        gath = _sc_gather(a.reshape(bsz * n, 128), idx_knm)
        return gath.reshape(bsz, KNN, n, 128), c

    g1, c1 = stage_gather(x, p['w1a'][:, :3], p['w1a'][:, 3:])
    x1, x1t = _run_q(g1, c1, _scale(p['g1a']), p['b1a'][None, :],
                     jnp.transpose(p['w1b']), _scale(p['g1b']),
                     p['b1b'][None, :])
    g2, c2 = stage_gather(x1t, p['w2a'][:, :64], p['w2a'][:, 64:])
    x2, x2t = _run_q(g2, c2, _scale(p['g2a']), p['b2a'][None, :],
                     jnp.transpose(p['w2b']), _scale(p['g2b']),
                     p['b2b'][None, :])
    g3, c3 = stage_gather(x2t, p['w3'][:, :64], p['w3'][:, 64:])
    x3, xmax = _run_qr1(g3, c3, _scale(p['g3']), p['b3'][None, :], x1, x2,
                        jnp.transpose(p['w4']), _scale(p['g4']),
                        p['b4'][None, :])
    out = _run_r2(xmax, l.reshape(bsz, 1, 16), jnp.transpose(p['wl']),
                  _scale(p['gl']), p['bl'][None, :],
                  jnp.transpose(p['w5'][:, :1088]),
                  x1, x2, x3, jnp.transpose(p['w5'][:, 1088:]),
                  _scale(p['g5']), p['b5'][None, :],
                  jnp.transpose(p['w6']), _scale(p['g6']), p['b6'][None, :],
                  jnp.transpose(p['w7']), _scale(p['g7']), p['b7'][None, :],
                  p['w8'], p['b8'][:, None])
    return out


# speedup vs baseline: 15.4884x; 1.0928x over previous
import functools
import jax
import jax.numpy as jnp
from jax import lax
from jax.experimental import pallas as pl
from jax.experimental.pallas import tpu as pltpu
from jax.experimental.pallas import tpu_sc as plsc

KNN = 20
NEG = -3.0e38


def _lrelu(y):
    return jnp.where(y >= 0, y, 0.2 * y)


def _p_body(nglob, xb_ref, xf_ref, wat_ref, wct_ref,
            idx_ref, a_ref, c_ref):
    b = pl.program_id(0)
    xb = xb_ref[0]
    xf = xf_ref[0]
    rb = xb.shape[1]
    n = xf.shape[1]
    inner = lax.dot_general(xb, xf, (((0,), (0,)), ((), ())),
                            preferred_element_type=jnp.float32)
    sqf = jnp.sum(xf * xf, axis=0, keepdims=True)
    sqb = jnp.transpose(jnp.sum(xb * xb, axis=0, keepdims=True))
    a_ref[0] = lax.dot_general(xb, wat_ref[...], (((0,), (0,)), ((), ())),
                               preferred_element_type=jnp.float32)
    c_ref[0] = lax.dot_general(xb, wct_ref[...], (((0,), (0,)), ((), ())),
                               preferred_element_type=jnp.float32)

    pair = (2.0 * inner - sqb) - sqf
    iota_j = lax.broadcasted_iota(jnp.int32, (rb, n), 1)
    lane_t = lax.broadcasted_iota(jnp.int32, (rb, 128), 1)
    acc = jnp.zeros((rb, 128), jnp.int32)
    base = b * nglob
    for t in range(KNN):
        m = jnp.max(pair, axis=1, keepdims=True)
        eqm = pair == m
        j = jnp.min(jnp.where(eqm, iota_j, n), axis=1, keepdims=True)
        acc = jnp.where(lane_t == t,
                        jnp.broadcast_to(j + base, (rb, 128)), acc)
        pair = jnp.where(eqm, NEG, pair)
    idx_ref[0] = acc


def _run_p(x, wat, wct, rb=256):
    bsz, ch, n = x.shape
    xb_spec = pl.BlockSpec((1, ch, rb), lambda b, r: (b, 0, r))
    xf_spec = pl.BlockSpec((1, ch, n), lambda b, r: (b, 0, 0))
    grid = (bsz, n // rb)
    out_shapes = (
        jax.ShapeDtypeStruct((bsz, n, 128), jnp.int32),
        jax.ShapeDtypeStruct((bsz, n, 128), jnp.float32),
        jax.ShapeDtypeStruct((bsz, n, 64), jnp.float32),
    )
    out_specs = (
        pl.BlockSpec((1, rb, 128), lambda b, r: (b, r, 0)),
        pl.BlockSpec((1, rb, 128), lambda b, r: (b, r, 0)),
        pl.BlockSpec((1, rb, 64), lambda b, r: (b, r, 0)),
    )
    wa_spec = pl.BlockSpec(wat.shape, lambda b, r: (0, 0))
    wc_spec = pl.BlockSpec(wct.shape, lambda b, r: (0, 0))
    return pl.pallas_call(
        functools.partial(_p_body, n),
        grid=grid,
        in_specs=[xb_spec, xf_spec, wa_spec, wc_spec],
        out_specs=out_specs,
        out_shape=out_shapes,
    )(x, x, wat, wct)


def _sc_gather(table, idx_flat):
    e_total = idx_flat.shape[0]
    info = plsc.get_sparse_core_info()
    nw = info.num_cores * info.num_subcores
    per_w = e_total // nw
    chunk = 128
    nchunk = per_w // chunk
    mesh = plsc.VectorSubcoreMesh(core_axis_name="c", subcore_axis_name="s")

    @functools.partial(
        pl.kernel, mesh=mesh,
        out_type=jax.ShapeDtypeStruct((e_total, 128), jnp.float32),
        scratch_types=[
            pltpu.VMEM((chunk,), jnp.int32),
            pltpu.VMEM((chunk, 128), jnp.float32),
            pltpu.SemaphoreType.DMA,
        ],
    )
    def gk(table_hbm, idx_hbm, out_hbm, idx_v, rows_v, sem):
        wid = lax.axis_index("s") * info.num_cores + lax.axis_index("c")
        wbase = wid * per_w

        def body(ci, carry):
            base = wbase + ci * chunk
            pltpu.sync_copy(idx_hbm.at[pl.ds(base, chunk)], idx_v)
            pltpu.async_copy(table_hbm.at[idx_v], rows_v, sem).wait()
            pltpu.sync_copy(rows_v, out_hbm.at[pl.ds(base, chunk)])
            return carry

        lax.fori_loop(0, nchunk, body, 0)

    return gk(table, idx_flat)


def _q_body(g_ref, c_ref, sa_ref, ba_ref, wbt_ref, sb_ref, bb_ref,
            out_ref, xt_ref):
    c = c_ref[0]
    sa = sa_ref[...]
    ba = ba_ref[...]
    hs = [
        _lrelu((g_ref[0, k][:, :64] + c) * sa + ba)
        for k in range(KNN)
    ]
    hcat = jnp.concatenate(hs, axis=0)
    y = jnp.dot(hcat, wbt_ref[...], preferred_element_type=jnp.float32)
    z = _lrelu(y * sb_ref[...] + bb_ref[...])
    rb = c.shape[0]
    acc = z[0:rb]
    for k in range(1, KNN):
        acc = jnp.maximum(acc, z[k * rb:(k + 1) * rb])
    out_ref[0] = acc
    xt_ref[0] = jnp.transpose(acc)


def _run_q(gath, cmat, sa, ba, wbt, sb, bb, rb=256):
    bsz, _, n, _ = gath.shape
    grid = (bsz, n // rb)
    small = lambda arr: pl.BlockSpec(arr.shape, lambda b, r: (0, 0))
    return pl.pallas_call(
        _q_body,
        grid=grid,
        in_specs=[
            pl.BlockSpec((1, KNN, rb, 128), lambda b, r: (b, 0, r, 0)),
            pl.BlockSpec((1, rb, 64), lambda b, r: (b, r, 0)),
            small(sa), small(ba), small(wbt), small(sb), small(bb),
        ],
        out_specs=(
            pl.BlockSpec((1, rb, 64), lambda b, r: (b, r, 0)),
            pl.BlockSpec((1, 64, rb), lambda b, r: (b, 0, r)),
        ),
        out_shape=(
            jax.ShapeDtypeStruct((bsz, n, 64), jnp.float32),
            jax.ShapeDtypeStruct((bsz, 64, n), jnp.float32),
        ),
    )(gath, cmat, sa, ba, wbt, sb, bb)


def _qr1_body(g_ref, c_ref, s3_ref, b3_ref, x1_ref, x2_ref, w4t_ref,
              s4_ref, b4_ref, x3_ref, xmax_ref):
    r = pl.program_id(1)
    mx = g_ref[0, 0][:, :64]
    for k in range(1, KNN):
        mx = jnp.maximum(mx, g_ref[0, k][:, :64])
    x3 = _lrelu((mx + c_ref[0]) * s3_ref[...] + b3_ref[...])
    x3_ref[0] = x3
    xc = jnp.concatenate([x1_ref[0], x2_ref[0], x3], axis=1)
    xg = _lrelu(jnp.dot(xc, w4t_ref[...], preferred_element_type=jnp.float32) * s4_ref[...] + b4_ref[...])
    part = jnp.max(xg, axis=0, keepdims=True)

    @pl.when(r == 0)
    def _():
        xmax_ref[0] = jnp.full((1, 1024), NEG, jnp.float32)

    xmax_ref[0] = jnp.maximum(xmax_ref[0], part)


def _run_qr1(gath3, c3, s3, b3, x1, x2, w4t, s4, b4, rb=256):
    bsz, _, n, _ = gath3.shape
    grid = (bsz, n // rb)
    small = lambda arr: pl.BlockSpec(arr.shape, lambda b, r: (0, 0))
    return pl.pallas_call(
        _qr1_body,
        grid=grid,
        in_specs=[
            pl.BlockSpec((1, KNN, rb, 128), lambda b, r: (b, 0, r, 0)),
            pl.BlockSpec((1, rb, 64), lambda b, r: (b, r, 0)),
            small(s3), small(b3),
            pl.BlockSpec((1, rb, 64), lambda b, r: (b, r, 0)),
            pl.BlockSpec((1, rb, 64), lambda b, r: (b, r, 0)),
            small(w4t), small(s4), small(b4),
        ],
        out_specs=(
            pl.BlockSpec((1, rb, 64), lambda b, r: (b, r, 0)),
            pl.BlockSpec((1, 1, 1024), lambda b, r: (b, 0, 0)),
        ),
        out_shape=(
            jax.ShapeDtypeStruct((bsz, n, 64), jnp.float32),
            jax.ShapeDtypeStruct((bsz, 1, 1024), jnp.float32),
        ),
    )(gath3, c3, s3, b3, x1, x2, w4t, s4, b4)


def _r2_body(xmax_ref, l_ref, wlt_ref, sl_ref, bl_ref, w5at_ref,
             x1_ref, x2_ref, x3_ref, w5bt_ref, s5_ref, b5_ref,
             w6t_ref, s6_ref, b6_ref, w7t_ref, s7_ref, b7_ref,
             w8_ref, b8_ref, out_ref):
    lc = _lrelu(jnp.dot(l_ref[0], wlt_ref[...],
                        preferred_element_type=jnp.float32) * sl_ref[...] + bl_ref[...])
    cin = jnp.concatenate([xmax_ref[0], lc], axis=1)
    const5 = jnp.dot(cin, w5at_ref[...], preferred_element_type=jnp.float32)
    xp = jnp.concatenate([x1_ref[0], x2_ref[0], x3_ref[0]], axis=1)
    y5 = _lrelu((jnp.dot(xp, w5bt_ref[...], preferred_element_type=jnp.float32) + const5) * s5_ref[...] + b5_ref[...])
    y6 = _lrelu(jnp.dot(y5, w6t_ref[...], preferred_element_type=jnp.float32) * s6_ref[...] + b6_ref[...])
    y7 = _lrelu(jnp.dot(y6, w7t_ref[...], preferred_element_type=jnp.float32) * s7_ref[...] + b7_ref[...])
    outt = lax.dot_general(w8_ref[...], y7, (((1,), (1,)), ((), ())),
                           preferred_element_type=jnp.float32)
    out_ref[0] = outt + b8_ref[...]


def _run_r2(xmax, l2, wlt, sl, bl, w5at, x1, x2, x3, w5bt, s5, b5,
            w6t, s6, b6, w7t, s7, b7, w8, b8c, rb=256):
    bsz, n, _ = x1.shape
    grid = (bsz, n // rb)
    small = lambda arr: pl.BlockSpec(arr.shape, lambda b, r: tuple(0 for _ in arr.shape))
    xspec = pl.BlockSpec((1, rb, 64), lambda b, r: (b, r, 0))
    return pl.pallas_call(
        _r2_body,
        grid=grid,
        in_specs=[
            pl.BlockSpec((1, 1, 1024), lambda b, r: (b, 0, 0)),
            pl.BlockSpec((1, 1, 16), lambda b, r: (b, 0, 0)),
            small(wlt), small(sl), small(bl), small(w5at),
            xspec, xspec, xspec,
            small(w5bt), small(s5), small(b5),
            small(w6t), small(s6), small(b6),
            small(w7t), small(s7), small(b7),
            small(w8), small(b8c),
        ],
        out_specs=pl.BlockSpec((1, 50, rb), lambda b, r: (b, 0, r)),
        out_shape=jax.ShapeDtypeStruct((bsz, 50, n), jnp.float32),
    )(xmax, l2, wlt, sl, bl, w5at, x1, x2, x3, w5bt, s5, b5,
      w6t, s6, b6, w7t, s7, b7, w8, b8c)


def _scale(g):
    return (g / jnp.sqrt(1.0 + 1e-5))[None, :]


def kernel(x, l, p):
    q = x.shape[0] // 4
    return jnp.concatenate([
        _forward(x[i * q:(i + 1) * q], l[i * q:(i + 1) * q], p)
        for i in range(4)
    ], axis=0)


def _forward(x, l, p):
    bsz, _, n = x.shape
    edges = bsz * KNN * n

    def stage_gather(xin, wa, wb):
        wat = jnp.pad(jnp.transpose(wa), ((0, 0), (0, 64)))
        wct = jnp.transpose(wb - wa)
        idx, a, c = _run_p(xin, wat, wct)
        idx_knm = jnp.transpose(idx[:, :, :KNN], (0, 2, 1)).reshape(edges)
        gath = _sc_gather(a.reshape(bsz * n, 128), idx_knm)
        return gath.reshape(bsz, KNN, n, 128), c

    g1, c1 = stage_gather(x, p['w1a'][:, :3], p['w1a'][:, 3:])
    x1, x1t = _run_q(g1, c1, _scale(p['g1a']), p['b1a'][None, :],
                     jnp.transpose(p['w1b']), _scale(p['g1b']),
                     p['b1b'][None, :])
    g2, c2 = stage_gather(x1t, p['w2a'][:, :64], p['w2a'][:, 64:])
    x2, x2t = _run_q(g2, c2, _scale(p['g2a']), p['b2a'][None, :],
                     jnp.transpose(p['w2b']), _scale(p['g2b']),
                     p['b2b'][None, :])
    g3, c3 = stage_gather(x2t, p['w3'][:, :64], p['w3'][:, 64:])
    x3, xmax = _run_qr1(g3, c3, _scale(p['g3']), p['b3'][None, :], x1, x2,
                        jnp.transpose(p['w4']), _scale(p['g4']),
                        p['b4'][None, :])
    out = _run_r2(xmax, l.reshape(bsz, 1, 16), jnp.transpose(p['wl']),
                  _scale(p['gl']), p['bl'][None, :],
                  jnp.transpose(p['w5'][:, :1088]),
                  x1, x2, x3, jnp.transpose(p['w5'][:, 1088:]),
                  _scale(p['g5']), p['b5'][None, :],
                  jnp.transpose(p['w6']), _scale(p['g6']), p['b6'][None, :],
                  jnp.transpose(p['w7']), _scale(p['g7']), p['b7'][None, :],
                  p['w8'], p['b8'][:, None])
    return out
